# baseline (speedup 1.0000x reference)
import functools

import jax
import jax.numpy as jnp
from jax import lax
from jax.experimental import pallas as pl
from jax.experimental.pallas import tpu as pltpu
from jax.experimental.pallas import tpu_sc as plsc

N = 10000
E = 160000
IN = 256
HID = 512
CLS = 128
CW = 128
NC = 2
NS = 16
NPAD = 10240
RPT = NPAD // NS
EPT = E // NS
K = 125
NB = EPT // K
BN = 1024
GRID = NPAD // BN

_MESH = plsc.VectorSubcoreMesh(
    core_axis_name="c", subcore_axis_name="s", num_cores=NC, num_subcores=NS)
_HI = jax.lax.Precision.HIGHEST
_f32 = jnp.float32


def _deg_body(idx_hbm, zeros_hbm, ones_hbm, out_hbm, acc, idx_v, ones_v,
              sem):
    c = lax.axis_index("c")
    s = lax.axis_index("s")
    pltpu.sync_copy(ones_hbm, ones_v)
    for dd in range(2):
        d = c * 2 + dd
        pltpu.sync_copy(zeros_hbm.at[pl.ds(s * RPT, RPT)],
                        acc.at[pl.ds(s * RPT, RPT)])
        pltpu.sync_copy(idx_hbm.at[d, s], idx_v)
        plsc.subcore_barrier()

        pltpu.async_copy(ones_v, acc.at[idx_v.at[0]], sem, add=True)

        def blk(j, carry):
            pltpu.async_copy(ones_v, acc.at[idx_v.at[j]], sem, add=True)
            pltpu.make_async_copy(ones_v, acc.at[idx_v.at[j - 1]],
                                  sem).wait()
            return carry

        lax.fori_loop(1, NB, blk, 0)
        pltpu.make_async_copy(ones_v, acc.at[idx_v.at[NB - 1]], sem).wait()
        plsc.subcore_barrier()
        pltpu.sync_copy(acc.at[pl.ds(s * RPT, RPT)],
                        out_hbm.at[d, pl.ds(s * RPT, RPT)])
        plsc.subcore_barrier()


_deg_kernel = pl.kernel(
    _deg_body,
    out_type=jax.ShapeDtypeStruct((4, NPAD, CW), _f32),
    mesh=_MESH,
    scratch_types=[
        pltpu.VMEM_SHARED((NPAD, CW), _f32),
        pltpu.VMEM((NB, K), jnp.int32),
        pltpu.VMEM((K, CW), _f32),
        pltpu.SemaphoreType.DMA,
    ],
)


def _spmm_body(C, x_hbm, src_hbm, dst_hbm, zeros_hbm, out_hbm,
               acc, src_v, dst_v, rows_a, rows_b,
               gsem_a, gsem_b, ssem_a, ssem_b):
    c = lax.axis_index("c")
    s = lax.axis_index("s")
    NH = NB // 2
    cpc = C // NC
    if True:
        pltpu.sync_copy(dst_hbm.at[s], dst_v)
        for cc in range(cpc):
            ch = c * cpc + cc
            pltpu.sync_copy(zeros_hbm.at[pl.ds(s * RPT, RPT)],
                            acc.at[pl.ds(s * RPT, RPT)])
            plsc.subcore_barrier()

            for half in range(2):
                base = half * NH
                pltpu.sync_copy(src_hbm.at[ch * NS + s, pl.ds(base, NH)],
                                src_v)
                pltpu.async_copy(x_hbm.at[src_v.at[0]], rows_a, gsem_a)
                pltpu.async_copy(x_hbm.at[src_v.at[1]], rows_b, gsem_b)

                def blk(jj, carry):
                    j = 2 * jj
                    j1 = j + 1
                    j2 = jnp.minimum(j + 2, NH - 1)
                    j3 = jnp.minimum(j + 3, NH - 1)
                    pltpu.make_async_copy(
                        x_hbm.at[src_v.at[j]], rows_a, gsem_a).wait()
                    pltpu.async_copy(rows_a, acc.at[dst_v.at[base + j]],
                                     ssem_a, add=True)
                    pltpu.make_async_copy(
                        x_hbm.at[src_v.at[j1]], rows_b, gsem_b).wait()
                    pltpu.async_copy(rows_b, acc.at[dst_v.at[base + j1]],
                                     ssem_b, add=True)
                    pltpu.make_async_copy(
                        rows_a, acc.at[dst_v.at[base + j]], ssem_a).wait()
                    pltpu.async_copy(x_hbm.at[src_v.at[j2]], rows_a, gsem_a)
                    pltpu.make_async_copy(
                        rows_b, acc.at[dst_v.at[base + j1]], ssem_b).wait()
                    pltpu.async_copy(x_hbm.at[src_v.at[j3]], rows_b, gsem_b)
                    return carry

                lax.fori_loop(0, NH // 2, blk, 0)
                pltpu.make_async_copy(x_hbm.at[src_v.at[NH - 1]], rows_a,
                                      gsem_a).wait()
                pltpu.make_async_copy(x_hbm.at[src_v.at[NH - 1]], rows_b,
                                      gsem_b).wait()
            plsc.subcore_barrier()
            pltpu.sync_copy(acc.at[pl.ds(s * RPT, RPT)],
                            out_hbm.at[pl.ds(ch * NPAD + s * RPT, RPT)])
            plsc.subcore_barrier()

def _make_spmm(C):
    return pl.kernel(
        functools.partial(_spmm_body, C),
        out_type=jax.ShapeDtypeStruct((C * NPAD, CW), _f32),
        mesh=_MESH,
        scratch_types=[
            pltpu.VMEM_SHARED((NPAD, CW), _f32),
            pltpu.VMEM((NB // 2, K), jnp.int32),
            pltpu.VMEM((NB, K), jnp.int32),
            pltpu.VMEM((K, CW), _f32),
            pltpu.VMEM((K, CW), _f32),
            pltpu.SemaphoreType.DMA,
            pltpu.SemaphoreType.DMA,
            pltpu.SemaphoreType.DMA,
            pltpu.SemaphoreType.DMA,
        ],
    )


_spmm2 = _make_spmm(2)
_spmm4 = _make_spmm(4)


def _norm_body(d_ref, o_ref):
    d = d_ref[...]
    o_ref[...] = jnp.where(d > 0, lax.rsqrt(d), 0.0)


def _norms(degs):
    return pl.pallas_call(
        _norm_body,
        out_shape=jax.ShapeDtypeStruct((4, NPAD, 16), _f32),
    )(degs)


def _prep0_body(f_ref, ns_ref, o_ref):
    t = f_ref[...] * ns_ref[:, 0:1]
    o_ref[0] = t[:, :CW]
    o_ref[1] = t[:, CW:]


def _prep0(featp, ns):
    return pl.pallas_call(
        _prep0_body,
        grid=(GRID,),
        in_specs=[
            pl.BlockSpec((BN, IN), lambda i: (i, 0)),
            pl.BlockSpec((BN, 16), lambda i: (i, 0)),
        ],
        out_specs=pl.BlockSpec((2, BN, CW), lambda i: (0, i, 0)),
        out_shape=jax.ShapeDtypeStruct((2, NPAD, CW), _f32),
    )(featp, ns)


def _layer1_body(a_ref, nd_ref, ns_ref, w_ref, o_ref):
    x = jnp.concatenate([a_ref[0], a_ref[1]], axis=-1)
    x = x * nd_ref[:, 0:1]
    h = jnp.maximum(
        jnp.dot(x, w_ref[...], preferred_element_type=_f32),
        0.0)
    t = h * ns_ref[:, 0:1]
    for c in range(4):
        o_ref[c] = t[:, c * CW:(c + 1) * CW]


def _layer1(agg0, nd, ns, W0):
    return pl.pallas_call(
        _layer1_body,
        grid=(GRID,),
        in_specs=[
            pl.BlockSpec((2, BN, CW), lambda i: (0, i, 0)),
            pl.BlockSpec((BN, 16), lambda i: (i, 0)),
            pl.BlockSpec((BN, 16), lambda i: (i, 0)),
            pl.BlockSpec((IN, HID), lambda i: (0, 0)),
        ],
        out_specs=pl.BlockSpec((4, BN, CW), lambda i: (0, i, 0)),
        out_shape=jax.ShapeDtypeStruct((4, NPAD, CW), _f32),
    )(agg0, nd, ns, W0)


def _layer2_body(a_ref, nd_ref, ns_ref, w1_ref, wc_ref, o_ref):
    x = jnp.concatenate([a_ref[i] for i in range(4)], axis=-1)
    x = x * nd_ref[:, 0:1]
    h = jnp.maximum(
        jnp.dot(x, w1_ref[...], preferred_element_type=_f32),
        0.0)
    t = jnp.dot(h * ns_ref[:, 0:1], wc_ref[...],
                preferred_element_type=_f32)
    o_ref[0] = t[:, :CW]
    o_ref[1] = t[:, CW:]


def _layer2(agg1, nd, ns, W1, Wcat):
    return pl.pallas_call(
        _layer2_body,
        grid=(GRID,),
        in_specs=[
            pl.BlockSpec((4, BN, CW), lambda i: (0, i, 0)),
            pl.BlockSpec((BN, 16), lambda i: (i, 0)),
            pl.BlockSpec((BN, 16), lambda i: (i, 0)),
            pl.BlockSpec((HID, HID), lambda i: (0, 0)),
            pl.BlockSpec((HID, 2 * CLS), lambda i: (0, 0)),
        ],
        out_specs=pl.BlockSpec((2, BN, CW), lambda i: (0, i, 0)),
        out_shape=jax.ShapeDtypeStruct((2, NPAD, CW), _f32),
    )(agg1, nd, ns, W1, Wcat)


def _final_body(a_ref, nd_ref, nz_ref, w_ref, z_ref, m_ref, sd_ref):
    ndv = nd_ref[0:N, 0:1]
    mean = a_ref[0, 0:N, :] * ndv
    stdv = a_ref[1, 0:N, :] * ndv
    zz = mean + nz_ref[...] * jnp.exp(stdv)
    mu = jnp.mean(zz, axis=0, keepdims=True)
    d = zz - mu
    var = jnp.sum(d * d, axis=0, keepdims=True) / (N - 1)
    zs = d * lax.rsqrt(var)
    z = lax.dot_general(zs, w_ref[...], (((1,), (1,)), ((), ())),
                        preferred_element_type=_f32)
    z_ref[...] = z
    m_ref[...] = mean
    sd_ref[...] = stdv


def _final(agg2, nd, noise, W_Z):
    return pl.pallas_call(
        _final_body,
        out_shape=(
            jax.ShapeDtypeStruct((N, CLS), _f32),
            jax.ShapeDtypeStruct((N, CLS), _f32),
            jax.ShapeDtypeStruct((N, CLS), _f32),
        ),
    )(agg2, nd, noise, W_Z)


def _mk_src(src, C):
    off = (jnp.arange(C, dtype=jnp.int32) * NPAD)[:, None]
    return (src[None, :] + off).reshape(C * NS, NB, K)


def kernel(g_edge_index, adj_edge_index, features, W0, W1, W2, W_std,
           W_Za, W_Zb, noise_A, noise_S):
    srcA, dstA = adj_edge_index[0], adj_edge_index[1]
    srcS, dstS = g_edge_index[0], g_edge_index[1]

    idx4 = jnp.stack([srcA, dstA, srcS, dstS]).reshape(4, NS, NB, K)
    zerosCW = jnp.zeros((NPAD, CW), _f32)
    ones_blk = jnp.ones((K, CW), _f32)
    degs = _deg_kernel(idx4, zerosCW, ones_blk)
    norms = _norms(degs[:, :, :16])
    nsA, ndA, nsS, ndS = norms[0], norms[1], norms[2], norms[3]

    featp = jnp.pad(features, ((0, NPAD - N), (0, 0)))
    Wcat = jnp.concatenate([W2, W_std], axis=1)

    dstA16 = dstA.reshape(NS, NB, K)
    dstS16 = dstS.reshape(NS, NB, K)
    srcA2, srcA4 = _mk_src(srcA, 2), _mk_src(srcA, 4)
    srcS2, srcS4 = _mk_src(srcS, 2), _mk_src(srcS, 4)

    t0A = _prep0(featp, nsA)
    t0S = _prep0(featp, nsS)
    aggA0 = _spmm2(t0A.reshape(2 * NPAD, CW), srcA2, dstA16, zerosCW)
    aggS0 = _spmm2(t0S.reshape(2 * NPAD, CW), srcS2, dstS16, zerosCW)
    t1A = _layer1(aggA0.reshape(2, NPAD, CW), ndA, nsA, W0)
    t1S = _layer1(aggS0.reshape(2, NPAD, CW), ndS, nsS, W0)
    aggA1 = _spmm4(t1A.reshape(4 * NPAD, CW), srcA4, dstA16, zerosCW)
    aggS1 = _spmm4(t1S.reshape(4 * NPAD, CW), srcS4, dstS16, zerosCW)
    t2A = _layer2(aggA1.reshape(4, NPAD, CW), ndA, nsA, W1, Wcat)
    t2S = _layer2(aggS1.reshape(4, NPAD, CW), ndS, nsS, W1, Wcat)
    aggA2 = _spmm2(t2A.reshape(2 * NPAD, CW), srcA2, dstA16, zerosCW)
    aggS2 = _spmm2(t2S.reshape(2 * NPAD, CW), srcS2, dstS16, zerosCW)
    z1, A_mean, A_std = _final(aggA2.reshape(2, NPAD, CW), ndA, noise_A, W_Za)
    z2, S_mean, S_std = _final(aggS2.reshape(2, NPAD, CW), ndS, noise_S, W_Zb)
    return (z1, z2, A_mean, S_mean, A_std, S_std)


# speedup vs baseline: 1.2695x; 1.2695x over previous
import functools

import jax
import jax.numpy as jnp
from jax import lax
from jax.experimental import pallas as pl
from jax.experimental.pallas import tpu as pltpu
from jax.experimental.pallas import tpu_sc as plsc

N = 10000
E = 160000
IN = 256
HID = 512
CLS = 128
CW = 128
NC = 2
NS = 16
NPAD = 10240
RPT = NPAD // NS
EPT = E // NS
K = 125
NB = EPT // K
BN = 1024
GRID = NPAD // BN

_MESH = plsc.VectorSubcoreMesh(
    core_axis_name="c", subcore_axis_name="s", num_cores=NC, num_subcores=NS)
_f32 = jnp.float32


def _deg_body(idx_hbm, zeros_hbm, ones_hbm, out_hbm, acc, idx_v, ones_v,
              sem):
    c = lax.axis_index("c")
    s = lax.axis_index("s")
    pltpu.sync_copy(ones_hbm, ones_v)
    for dd in range(2):
        d = c * 2 + dd
        pltpu.sync_copy(zeros_hbm.at[pl.ds(s * RPT, RPT)],
                        acc.at[pl.ds(s * RPT, RPT)])
        pltpu.sync_copy(idx_hbm.at[d, s], idx_v)
        plsc.subcore_barrier()

        pltpu.async_copy(ones_v, acc.at[idx_v.at[0]], sem, add=True)

        def blk(j, carry):
            pltpu.async_copy(ones_v, acc.at[idx_v.at[j]], sem, add=True)
            pltpu.make_async_copy(ones_v, acc.at[idx_v.at[j - 1]],
                                  sem).wait()
            return carry

        lax.fori_loop(1, NB, blk, 0)
        pltpu.make_async_copy(ones_v, acc.at[idx_v.at[NB - 1]], sem).wait()
        plsc.subcore_barrier()
        pltpu.sync_copy(acc.at[pl.ds(s * RPT, RPT)],
                        out_hbm.at[d, pl.ds(s * RPT, RPT)])
        plsc.subcore_barrier()


_deg_kernel = pl.kernel(
    _deg_body,
    out_type=jax.ShapeDtypeStruct((4, NPAD, CW), _f32),
    mesh=_MESH,
    scratch_types=[
        pltpu.VMEM_SHARED((NPAD, CW), _f32),
        pltpu.VMEM((NB, K), jnp.int32),
        pltpu.VMEM((K, CW), _f32),
        pltpu.SemaphoreType.DMA,
    ],
)


def _spmm_body(C, x_hbm, src_hbm, dst_hbm, zeros_hbm, out_hbm,
               acc, src_v, dst_v, rows_a, rows_b, gsem_a, gsem_b):
    c = lax.axis_index("c")
    s = lax.axis_index("s")
    NH = NB // 2
    cpc = C // NC
    if True:
        pltpu.sync_copy(dst_hbm.at[s], dst_v)
        for cc in range(cpc):
            ch = c * cpc + cc
            pltpu.sync_copy(zeros_hbm.at[pl.ds(s * RPT, RPT)],
                            acc.at[pl.ds(s * RPT, RPT)])
            plsc.subcore_barrier()

            for half in range(2):
                base = half * NH
                pltpu.sync_copy(src_hbm.at[ch * NS + s, pl.ds(base, NH)],
                                src_v)
                pltpu.async_copy(x_hbm.at[src_v.at[0]], rows_a, gsem_a)

                def blk(jj, carry):
                    j = 2 * jj
                    j1 = jnp.minimum(j + 1, NH - 1)
                    j2 = jnp.minimum(j + 2, NH - 1)
                    pltpu.async_copy(x_hbm.at[src_v.at[j1]], rows_b, gsem_b)
                    pltpu.make_async_copy(
                        x_hbm.at[src_v.at[j]], rows_a, gsem_a).wait()
                    pltpu.sync_copy(rows_a, acc.at[dst_v.at[base + j]],
                                    add=True)
                    pltpu.async_copy(x_hbm.at[src_v.at[j2]], rows_a, gsem_a)
                    pltpu.make_async_copy(
                        x_hbm.at[src_v.at[j1]], rows_b, gsem_b).wait()
                    pltpu.sync_copy(rows_b, acc.at[dst_v.at[base + j1]],
                                    add=True)
                    return carry

                lax.fori_loop(0, NH // 2, blk, 0)
                pltpu.make_async_copy(x_hbm.at[src_v.at[NH - 1]], rows_a,
                                      gsem_a).wait()
            plsc.subcore_barrier()
            pltpu.sync_copy(acc.at[pl.ds(s * RPT, RPT)],
                            out_hbm.at[pl.ds(ch * NPAD + s * RPT, RPT)])
            plsc.subcore_barrier()

def _make_spmm(C):
    return pl.kernel(
        functools.partial(_spmm_body, C),
        out_type=jax.ShapeDtypeStruct((C * NPAD, CW), _f32),
        mesh=_MESH,
        scratch_types=[
            pltpu.VMEM_SHARED((NPAD, CW), _f32),
            pltpu.VMEM((NB // 2, K), jnp.int32),
            pltpu.VMEM((NB, K), jnp.int32),
            pltpu.VMEM((K, CW), _f32),
            pltpu.VMEM((K, CW), _f32),
            pltpu.SemaphoreType.DMA,
            pltpu.SemaphoreType.DMA,
        ],
    )


_spmm2 = _make_spmm(2)
_spmm4 = _make_spmm(4)


def _norm_body(d_ref, o_ref):
    d = d_ref[...]
    o_ref[...] = jnp.where(d > 0, lax.rsqrt(d), 0.0)


def _norms(degs):
    return pl.pallas_call(
        _norm_body,
        out_shape=jax.ShapeDtypeStruct((4, NPAD, 16), _f32),
    )(degs)


def _prep0_body(f_ref, ns_ref, o_ref):
    t = f_ref[...] * ns_ref[:, 0:1]
    o_ref[0] = t[:, :CW]
    o_ref[1] = t[:, CW:]


def _prep0(featp, ns):
    return pl.pallas_call(
        _prep0_body,
        grid=(GRID,),
        in_specs=[
            pl.BlockSpec((BN, IN), lambda i: (i, 0)),
            pl.BlockSpec((BN, 16), lambda i: (i, 0)),
        ],
        out_specs=pl.BlockSpec((2, BN, CW), lambda i: (0, i, 0)),
        out_shape=jax.ShapeDtypeStruct((2, NPAD, CW), _f32),
    )(featp, ns)


def _layer1_body(a_ref, nd_ref, ns_ref, w_ref, o_ref):
    x = jnp.concatenate([a_ref[0], a_ref[1]], axis=-1)
    x = x * nd_ref[:, 0:1]
    h = jnp.maximum(
        jnp.dot(x, w_ref[...], preferred_element_type=_f32),
        0.0)
    t = h * ns_ref[:, 0:1]
    for c in range(4):
        o_ref[c] = t[:, c * CW:(c + 1) * CW]


def _layer1(agg0, nd, ns, W0):
    return pl.pallas_call(
        _layer1_body,
        grid=(GRID,),
        in_specs=[
            pl.BlockSpec((2, BN, CW), lambda i: (0, i, 0)),
            pl.BlockSpec((BN, 16), lambda i: (i, 0)),
            pl.BlockSpec((BN, 16), lambda i: (i, 0)),
            pl.BlockSpec((IN, HID), lambda i: (0, 0)),
        ],
        out_specs=pl.BlockSpec((4, BN, CW), lambda i: (0, i, 0)),
        out_shape=jax.ShapeDtypeStruct((4, NPAD, CW), _f32),
    )(agg0, nd, ns, W0)


def _layer2_body(a_ref, nd_ref, ns_ref, w1_ref, wc_ref, o_ref):
    x = jnp.concatenate([a_ref[i] for i in range(4)], axis=-1)
    x = x * nd_ref[:, 0:1]
    h = jnp.maximum(
        jnp.dot(x, w1_ref[...], preferred_element_type=_f32),
        0.0)
    t = jnp.dot(h * ns_ref[:, 0:1], wc_ref[...],
                preferred_element_type=_f32)
    o_ref[0] = t[:, :CW]
    o_ref[1] = t[:, CW:]


def _layer2(agg1, nd, ns, W1, Wcat):
    return pl.pallas_call(
        _layer2_body,
        grid=(GRID,),
        in_specs=[
            pl.BlockSpec((4, BN, CW), lambda i: (0, i, 0)),
            pl.BlockSpec((BN, 16), lambda i: (i, 0)),
            pl.BlockSpec((BN, 16), lambda i: (i, 0)),
            pl.BlockSpec((HID, HID), lambda i: (0, 0)),
            pl.BlockSpec((HID, 2 * CLS), lambda i: (0, 0)),
        ],
        out_specs=pl.BlockSpec((2, BN, CW), lambda i: (0, i, 0)),
        out_shape=jax.ShapeDtypeStruct((2, NPAD, CW), _f32),
    )(agg1, nd, ns, W1, Wcat)


def _final_body(a_ref, nd_ref, nz_ref, w_ref, z_ref, m_ref, sd_ref):
    ndv = nd_ref[0:N, 0:1]
    mean = a_ref[0, 0:N, :] * ndv
    stdv = a_ref[1, 0:N, :] * ndv
    zz = mean + nz_ref[...] * jnp.exp(stdv)
    mu = jnp.mean(zz, axis=0, keepdims=True)
    d = zz - mu
    var = jnp.sum(d * d, axis=0, keepdims=True) / (N - 1)
    zs = d * lax.rsqrt(var)
    z = lax.dot_general(zs, w_ref[...], (((1,), (1,)), ((), ())),
                        preferred_element_type=_f32)
    z_ref[...] = z
    m_ref[...] = mean
    sd_ref[...] = stdv


def _final(agg2, nd, noise, W_Z):
    return pl.pallas_call(
        _final_body,
        out_shape=(
            jax.ShapeDtypeStruct((N, CLS), _f32),
            jax.ShapeDtypeStruct((N, CLS), _f32),
            jax.ShapeDtypeStruct((N, CLS), _f32),
        ),
    )(agg2, nd, noise, W_Z)


def _mk_src(src, C):
    off = (jnp.arange(C, dtype=jnp.int32) * NPAD)[:, None]
    return (src[None, :] + off).reshape(C * NS, NB, K)


def kernel(g_edge_index, adj_edge_index, features, W0, W1, W2, W_std,
           W_Za, W_Zb, noise_A, noise_S):
    srcA, dstA = adj_edge_index[0], adj_edge_index[1]
    srcS, dstS = g_edge_index[0], g_edge_index[1]

    idx4 = jnp.stack([srcA, dstA, srcS, dstS]).reshape(4, NS, NB, K)
    zerosCW = jnp.zeros((NPAD, CW), _f32)
    ones_blk = jnp.ones((K, CW), _f32)
    degs = _deg_kernel(idx4, zerosCW, ones_blk)
    norms = _norms(degs[:, :, :16])
    nsA, ndA, nsS, ndS = norms[0], norms[1], norms[2], norms[3]

    featp = jnp.pad(features, ((0, NPAD - N), (0, 0)))
    Wcat = jnp.concatenate([W2, W_std], axis=1)

    dstA16 = dstA.reshape(NS, NB, K)
    dstS16 = dstS.reshape(NS, NB, K)
    srcA2, srcA4 = _mk_src(srcA, 2), _mk_src(srcA, 4)
    srcS2, srcS4 = _mk_src(srcS, 2), _mk_src(srcS, 4)

    t0A = _prep0(featp, nsA)
    t0S = _prep0(featp, nsS)
    aggA0 = _spmm2(t0A.reshape(2 * NPAD, CW), srcA2, dstA16, zerosCW)
    aggS0 = _spmm2(t0S.reshape(2 * NPAD, CW), srcS2, dstS16, zerosCW)
    t1A = _layer1(aggA0.reshape(2, NPAD, CW), ndA, nsA, W0)
    t1S = _layer1(aggS0.reshape(2, NPAD, CW), ndS, nsS, W0)
    aggA1 = _spmm4(t1A.reshape(4 * NPAD, CW), srcA4, dstA16, zerosCW)
    aggS1 = _spmm4(t1S.reshape(4 * NPAD, CW), srcS4, dstS16, zerosCW)
    t2A = _layer2(aggA1.reshape(4, NPAD, CW), ndA, nsA, W1, Wcat)
    t2S = _layer2(aggS1.reshape(4, NPAD, CW), ndS, nsS, W1, Wcat)
    aggA2 = _spmm2(t2A.reshape(2 * NPAD, CW), srcA2, dstA16, zerosCW)
    aggS2 = _spmm2(t2S.reshape(2 * NPAD, CW), srcS2, dstS16, zerosCW)
    z1, A_mean, A_std = _final(aggA2.reshape(2, NPAD, CW), ndA, noise_A, W_Za)
    z2, S_mean, S_std = _final(aggS2.reshape(2, NPAD, CW), ndS, noise_S, W_Zb)
    return (z1, z2, A_mean, S_mean, A_std, S_std)


# speedup vs baseline: 1.2710x; 1.0012x over previous
import functools

import jax
import jax.numpy as jnp
from jax import lax
from jax.experimental import pallas as pl
from jax.experimental.pallas import tpu as pltpu
from jax.experimental.pallas import tpu_sc as plsc

N = 10000
E = 160000
IN = 256
HID = 512
CLS = 128
CW = 128
NC = 2
NS = 16
NPAD = 10240
RPT = NPAD // NS
EPT = E // NS
K = 125
NB = EPT // K
BN = 1024
GRID = NPAD // BN

_MESH = plsc.VectorSubcoreMesh(
    core_axis_name="c", subcore_axis_name="s", num_cores=NC, num_subcores=NS)
_f32 = jnp.float32


def _deg_body(idx_hbm, zeros_hbm, ones_hbm, out_hbm, acc, idx_v, ones_v,
              sem):
    c = lax.axis_index("c")
    s = lax.axis_index("s")
    pltpu.sync_copy(ones_hbm, ones_v)
    for dd in range(2):
        d = c * 2 + dd
        pltpu.sync_copy(zeros_hbm.at[pl.ds(s * RPT, RPT)],
                        acc.at[pl.ds(s * RPT, RPT)])
        pltpu.sync_copy(idx_hbm.at[d, s], idx_v)
        plsc.subcore_barrier()

        pltpu.async_copy(ones_v, acc.at[idx_v.at[0]], sem, add=True)

        def blk(j, carry):
            pltpu.async_copy(ones_v, acc.at[idx_v.at[j]], sem, add=True)
            pltpu.make_async_copy(ones_v, acc.at[idx_v.at[j - 1]],
                                  sem).wait()
            return carry

        lax.fori_loop(1, NB, blk, 0)
        pltpu.make_async_copy(ones_v, acc.at[idx_v.at[NB - 1]], sem).wait()
        plsc.subcore_barrier()
        pltpu.sync_copy(acc.at[pl.ds(s * RPT, RPT)],
                        out_hbm.at[d, pl.ds(s * RPT, RPT)])
        plsc.subcore_barrier()


_deg_kernel = pl.kernel(
    _deg_body,
    out_type=jax.ShapeDtypeStruct((4, NPAD, CW), _f32),
    mesh=_MESH,
    scratch_types=[
        pltpu.VMEM_SHARED((NPAD, CW), _f32),
        pltpu.VMEM((NB, K), jnp.int32),
        pltpu.VMEM((K, CW), _f32),
        pltpu.SemaphoreType.DMA,
    ],
)


def _spmm_body(C, x_hbm, src_hbm, dst_hbm, zeros_hbm, out_hbm,
               acc, src_v, dst_v, rows_a, rows_b, gsem_a, gsem_b):
    c = lax.axis_index("c")
    s = lax.axis_index("s")
    NH = NB // 2
    cpc = C // NC
    pltpu.sync_copy(dst_hbm.at[s], dst_v)
    for cc in range(cpc):
        ch = c * cpc + cc
        pltpu.sync_copy(zeros_hbm.at[pl.ds(s * RPT, RPT)],
                        acc.at[pl.ds(s * RPT, RPT)])
        plsc.subcore_barrier()

        for half in range(2):
            base = half * NH
            pltpu.sync_copy(src_hbm.at[ch * NS + s, pl.ds(base, NH)],
                            src_v)
            pltpu.async_copy(x_hbm.at[src_v.at[0]], rows_a, gsem_a)

            def blk(jj, carry):
                j = 2 * jj
                j1 = jnp.minimum(j + 1, NH - 1)
                j2 = jnp.minimum(j + 2, NH - 1)
                pltpu.async_copy(x_hbm.at[src_v.at[j1]], rows_b, gsem_b)
                pltpu.make_async_copy(
                    x_hbm.at[src_v.at[j]], rows_a, gsem_a).wait()
                pltpu.sync_copy(rows_a, acc.at[dst_v.at[base + j]],
                                add=True)
                pltpu.async_copy(x_hbm.at[src_v.at[j2]], rows_a, gsem_a)
                pltpu.make_async_copy(
                    x_hbm.at[src_v.at[j1]], rows_b, gsem_b).wait()
                pltpu.sync_copy(rows_b, acc.at[dst_v.at[base + j1]],
                                add=True)
                return carry

            lax.fori_loop(0, NH // 2, blk, 0)
            pltpu.make_async_copy(x_hbm.at[src_v.at[NH - 1]], rows_a,
                                  gsem_a).wait()
        plsc.subcore_barrier()
        pltpu.sync_copy(acc.at[pl.ds(s * RPT, RPT)],
                        out_hbm.at[pl.ds(ch * NPAD + s * RPT, RPT)])
        plsc.subcore_barrier()

def _make_spmm(C):
    return pl.kernel(
        functools.partial(_spmm_body, C),
        out_type=jax.ShapeDtypeStruct((C * NPAD, CW), _f32),
        mesh=_MESH,
        scratch_types=[
            pltpu.VMEM_SHARED((NPAD, CW), _f32),
            pltpu.VMEM((NB // 2, K), jnp.int32),
            pltpu.VMEM((NB, K), jnp.int32),
            pltpu.VMEM((K, CW), _f32),
            pltpu.VMEM((K, CW), _f32),
            pltpu.SemaphoreType.DMA,
            pltpu.SemaphoreType.DMA,
        ],
    )


_spmm2 = _make_spmm(2)
_spmm4 = _make_spmm(4)


def _norm_body(d_ref, o_ref):
    d = d_ref[...]
    o_ref[...] = jnp.where(d > 0, lax.rsqrt(d), 0.0)


def _norms(degs):
    return pl.pallas_call(
        _norm_body,
        out_shape=jax.ShapeDtypeStruct((4, NPAD, 16), _f32),
    )(degs)


def _prep0_body(f_ref, ns_ref, o_ref):
    t = f_ref[...] * ns_ref[:, 0:1]
    o_ref[0] = t[:, :CW]
    o_ref[1] = t[:, CW:]


def _prep0(featp, ns):
    return pl.pallas_call(
        _prep0_body,
        grid=(GRID,),
        in_specs=[
            pl.BlockSpec((BN, IN), lambda i: (i, 0)),
            pl.BlockSpec((BN, 16), lambda i: (i, 0)),
        ],
        out_specs=pl.BlockSpec((2, BN, CW), lambda i: (0, i, 0)),
        out_shape=jax.ShapeDtypeStruct((2, NPAD, CW), _f32),
    )(featp, ns)


def _layer1_body(a_ref, nd_ref, ns_ref, w_ref, o_ref):
    x = jnp.concatenate([a_ref[0], a_ref[1]], axis=-1)
    x = x * nd_ref[:, 0:1]
    h = jnp.maximum(
        jnp.dot(x, w_ref[...], preferred_element_type=_f32),
        0.0)
    t = h * ns_ref[:, 0:1]
    for c in range(4):
        o_ref[c] = t[:, c * CW:(c + 1) * CW]


def _layer1(agg0, nd, ns, W0):
    return pl.pallas_call(
        _layer1_body,
        grid=(GRID,),
        in_specs=[
            pl.BlockSpec((2, BN, CW), lambda i: (0, i, 0)),
            pl.BlockSpec((BN, 16), lambda i: (i, 0)),
            pl.BlockSpec((BN, 16), lambda i: (i, 0)),
            pl.BlockSpec((IN, HID), lambda i: (0, 0)),
        ],
        out_specs=pl.BlockSpec((4, BN, CW), lambda i: (0, i, 0)),
        out_shape=jax.ShapeDtypeStruct((4, NPAD, CW), _f32),
    )(agg0, nd, ns, W0)


def _layer2_body(a_ref, nd_ref, ns_ref, w1_ref, wc_ref, o_ref):
    x = jnp.concatenate([a_ref[i] for i in range(4)], axis=-1)
    x = x * nd_ref[:, 0:1]
    h = jnp.maximum(
        jnp.dot(x, w1_ref[...], preferred_element_type=_f32),
        0.0)
    t = jnp.dot(h * ns_ref[:, 0:1], wc_ref[...],
                preferred_element_type=_f32)
    o_ref[0] = t[:, :CW]
    o_ref[1] = t[:, CW:]


def _layer2(agg1, nd, ns, W1, Wcat):
    return pl.pallas_call(
        _layer2_body,
        grid=(GRID,),
        in_specs=[
            pl.BlockSpec((4, BN, CW), lambda i: (0, i, 0)),
            pl.BlockSpec((BN, 16), lambda i: (i, 0)),
            pl.BlockSpec((BN, 16), lambda i: (i, 0)),
            pl.BlockSpec((HID, HID), lambda i: (0, 0)),
            pl.BlockSpec((HID, 2 * CLS), lambda i: (0, 0)),
        ],
        out_specs=pl.BlockSpec((2, BN, CW), lambda i: (0, i, 0)),
        out_shape=jax.ShapeDtypeStruct((2, NPAD, CW), _f32),
    )(agg1, nd, ns, W1, Wcat)


def _final_body(a_ref, nd_ref, nz_ref, w_ref, z_ref, m_ref, sd_ref):
    ndv = nd_ref[0:N, 0:1]
    mean = a_ref[0, 0:N, :] * ndv
    stdv = a_ref[1, 0:N, :] * ndv
    zz = mean + nz_ref[...] * jnp.exp(stdv)
    mu = jnp.mean(zz, axis=0, keepdims=True)
    d = zz - mu
    var = jnp.sum(d * d, axis=0, keepdims=True) / (N - 1)
    zs = d * lax.rsqrt(var)
    z = lax.dot_general(zs, w_ref[...], (((1,), (1,)), ((), ())),
                        preferred_element_type=_f32)
    z_ref[...] = z
    m_ref[...] = mean
    sd_ref[...] = stdv


def _final(agg2, nd, noise, W_Z):
    return pl.pallas_call(
        _final_body,
        out_shape=(
            jax.ShapeDtypeStruct((N, CLS), _f32),
            jax.ShapeDtypeStruct((N, CLS), _f32),
            jax.ShapeDtypeStruct((N, CLS), _f32),
        ),
    )(agg2, nd, noise, W_Z)


def _mk_src(src, C):
    off = (jnp.arange(C, dtype=jnp.int32) * NPAD)[:, None]
    return (src[None, :] + off).reshape(C * NS, NB, K)


def kernel(g_edge_index, adj_edge_index, features, W0, W1, W2, W_std,
           W_Za, W_Zb, noise_A, noise_S):
    srcA, dstA = adj_edge_index[0], adj_edge_index[1]
    srcS, dstS = g_edge_index[0], g_edge_index[1]

    idx4 = jnp.stack([srcA, dstA, srcS, dstS]).reshape(4, NS, NB, K)
    zerosCW = jnp.zeros((NPAD, CW), _f32)
    ones_blk = jnp.ones((K, CW), _f32)
    degs = _deg_kernel(idx4, zerosCW, ones_blk)
    norms = _norms(degs[:, :, :16])
    nsA, ndA, nsS, ndS = norms[0], norms[1], norms[2], norms[3]

    featp = jnp.pad(features, ((0, NPAD - N), (0, 0)))
    Wcat = jnp.concatenate([W2, W_std], axis=1)

    dstA16 = dstA.reshape(NS, NB, K)
    dstS16 = dstS.reshape(NS, NB, K)
    srcA2, srcA4 = _mk_src(srcA, 2), _mk_src(srcA, 4)
    srcS2, srcS4 = _mk_src(srcS, 2), _mk_src(srcS, 4)

    t0A = _prep0(featp, nsA)
    t0S = _prep0(featp, nsS)
    aggA0 = _spmm2(t0A.reshape(2 * NPAD, CW), srcA2, dstA16, zerosCW)
    aggS0 = _spmm2(t0S.reshape(2 * NPAD, CW), srcS2, dstS16, zerosCW)
    t1A = _layer1(aggA0.reshape(2, NPAD, CW), ndA, nsA, W0)
    t1S = _layer1(aggS0.reshape(2, NPAD, CW), ndS, nsS, W0)
    aggA1 = _spmm4(t1A.reshape(4 * NPAD, CW), srcA4, dstA16, zerosCW)
    aggS1 = _spmm4(t1S.reshape(4 * NPAD, CW), srcS4, dstS16, zerosCW)
    t2A = _layer2(aggA1.reshape(4, NPAD, CW), ndA, nsA, W1, Wcat)
    t2S = _layer2(aggS1.reshape(4, NPAD, CW), ndS, nsS, W1, Wcat)
    aggA2 = _spmm2(t2A.reshape(2 * NPAD, CW), srcA2, dstA16, zerosCW)
    aggS2 = _spmm2(t2S.reshape(2 * NPAD, CW), srcS2, dstS16, zerosCW)
    z1, A_mean, A_std = _final(aggA2.reshape(2, NPAD, CW), ndA, noise_A, W_Za)
    z2, S_mean, S_std = _final(aggS2.reshape(2, NPAD, CW), ndS, noise_S, W_Zb)
    return (z1, z2, A_mean, S_mean, A_std, S_std)
